# baseline (speedup 1.0000x reference)
import functools

import jax
import jax.numpy as jnp
from jax import lax
from jax.experimental import pallas as pl
from jax.experimental.pallas import tpu as pltpu

_ROWS = 65536
_D = 256
_CH = 2048
_NCH = _ROWS // _CH
_NBUF = 8


def _ring_body(y_ref, c_ref, w_ref, out_ref, bufs, yv, in_sems, out_sems, ysem):
    def in_copy(k):
        return pltpu.make_async_copy(
            w_ref.at[pl.ds(k * _CH, _CH)], bufs.at[k % _NBUF],
            in_sems.at[k % _NBUF])

    def out_copy(k):
        return pltpu.make_async_copy(
            bufs.at[k % _NBUF], out_ref.at[pl.ds(k * _CH, _CH)],
            out_sems.at[k % _NBUF])

    ycp = pltpu.make_async_copy(y_ref.at[pl.ds(1, 3)], yv, ysem)
    ycp.start()
    for k in range(_NBUF):
        in_copy(k).start()
    ycp.wait()

    for k in range(_NCH):
        in_copy(k).wait()
        if k == 0:
            c17 = c_ref[17]
            c18 = c_ref[18]
            c19 = c_ref[19]
            bufs[0, 0:1, :] = jnp.full((1, _D), c19 * c17, jnp.float32)
            bufs[0, 1:2, :] = jnp.full((1, _D), c18 / c19, jnp.float32)
            bufs[0, 2:3, :] = yv[2:3, :] + yv[0:1, :] + 2.0 * yv[1:2, :]
        out_copy(k).start()
        if k + _NBUF < _NCH:
            out_copy(k).wait()
            in_copy(k + _NBUF).start()
    for k in range(max(0, _NCH - _NBUF), _NCH):
        out_copy(k).wait()


def kernel(y, w, c, t):
    del t
    return pl.pallas_call(
        _ring_body,
        out_shape=jax.ShapeDtypeStruct((_ROWS, _D), jnp.float32),
        in_specs=[
            pl.BlockSpec(memory_space=pl.ANY),
            pl.BlockSpec(memory_space=pltpu.SMEM),
            pl.BlockSpec(memory_space=pl.ANY),
        ],
        out_specs=pl.BlockSpec(memory_space=pl.ANY),
        scratch_shapes=[
            pltpu.VMEM((_NBUF, _CH, _D), jnp.float32),
            pltpu.VMEM((3, _D), jnp.float32),
            pltpu.SemaphoreType.DMA((_NBUF,)),
            pltpu.SemaphoreType.DMA((_NBUF,)),
            pltpu.SemaphoreType.DMA,
        ],
        compiler_params=pltpu.CompilerParams(
            vmem_limit_bytes=134217728,
        ),
    )(y, c, w)


# speedup vs baseline: 1.1394x; 1.1394x over previous
import functools

import jax
import jax.numpy as jnp
from jax import lax
from jax.experimental import pallas as pl
from jax.experimental.pallas import tpu as pltpu

_ROWS = 65536
_D = 256
_CH = 8192
_NCH = _ROWS // _CH
_NBUF = 4


def _ring_body(y_ref, c_ref, w_ref, out_ref, bufs, yv, in_sems, out_sems, ysem):
    def in_copy(k):
        return pltpu.make_async_copy(
            w_ref.at[pl.ds(k * _CH, _CH)], bufs.at[k % _NBUF],
            in_sems.at[k % _NBUF])

    def out_copy(k):
        return pltpu.make_async_copy(
            bufs.at[k % _NBUF], out_ref.at[pl.ds(k * _CH, _CH)],
            out_sems.at[k % _NBUF])

    ycp = pltpu.make_async_copy(y_ref.at[pl.ds(1, 3)], yv, ysem)
    ycp.start()
    for k in range(_NBUF):
        in_copy(k).start()
    ycp.wait()

    for k in range(_NCH):
        in_copy(k).wait()
        if k == 0:
            c17 = c_ref[17]
            c18 = c_ref[18]
            c19 = c_ref[19]
            bufs[0, 0:1, :] = jnp.full((1, _D), c19 * c17, jnp.float32)
            bufs[0, 1:2, :] = jnp.full((1, _D), c18 / c19, jnp.float32)
            bufs[0, 2:3, :] = yv[2:3, :] + yv[0:1, :] + 2.0 * yv[1:2, :]
        out_copy(k).start()
        if k + _NBUF < _NCH:
            out_copy(k).wait()
            in_copy(k + _NBUF).start()
    for k in range(max(0, _NCH - _NBUF), _NCH):
        out_copy(k).wait()


def kernel(y, w, c, t):
    del t
    return pl.pallas_call(
        _ring_body,
        out_shape=jax.ShapeDtypeStruct((_ROWS, _D), jnp.float32),
        in_specs=[
            pl.BlockSpec(memory_space=pl.ANY),
            pl.BlockSpec(memory_space=pltpu.SMEM),
            pl.BlockSpec(memory_space=pl.ANY),
        ],
        out_specs=pl.BlockSpec(memory_space=pl.ANY),
        scratch_shapes=[
            pltpu.VMEM((_NBUF, _CH, _D), jnp.float32),
            pltpu.VMEM((3, _D), jnp.float32),
            pltpu.SemaphoreType.DMA((_NBUF,)),
            pltpu.SemaphoreType.DMA((_NBUF,)),
            pltpu.SemaphoreType.DMA,
        ],
        compiler_params=pltpu.CompilerParams(
            vmem_limit_bytes=134217728,
        ),
    )(y, c, w)


# speedup vs baseline: 1.1552x; 1.0139x over previous
import functools

import jax
import jax.numpy as jnp
from jax import lax
from jax.experimental import pallas as pl
from jax.experimental.pallas import tpu as pltpu

_ROWS = 65536
_D = 256
_CH = 16384
_NCH = _ROWS // _CH
_NBUF = 3


def _ring_body(y_ref, c_ref, w_ref, out_ref, bufs, yv, in_sems, out_sems, ysem):
    def in_copy(k):
        return pltpu.make_async_copy(
            w_ref.at[pl.ds(k * _CH, _CH)], bufs.at[k % _NBUF],
            in_sems.at[k % _NBUF])

    def out_copy(k):
        return pltpu.make_async_copy(
            bufs.at[k % _NBUF], out_ref.at[pl.ds(k * _CH, _CH)],
            out_sems.at[k % _NBUF])

    ycp = pltpu.make_async_copy(y_ref.at[pl.ds(1, 3)], yv, ysem)
    ycp.start()
    for k in range(_NBUF):
        in_copy(k).start()
    ycp.wait()

    for k in range(_NCH):
        in_copy(k).wait()
        if k == 0:
            c17 = c_ref[17]
            c18 = c_ref[18]
            c19 = c_ref[19]
            bufs[0, 0:1, :] = jnp.full((1, _D), c19 * c17, jnp.float32)
            bufs[0, 1:2, :] = jnp.full((1, _D), c18 / c19, jnp.float32)
            bufs[0, 2:3, :] = yv[2:3, :] + yv[0:1, :] + 2.0 * yv[1:2, :]
        out_copy(k).start()
        if k + _NBUF < _NCH:
            out_copy(k).wait()
            in_copy(k + _NBUF).start()
    for k in range(max(0, _NCH - _NBUF), _NCH):
        out_copy(k).wait()


def kernel(y, w, c, t):
    del t
    return pl.pallas_call(
        _ring_body,
        out_shape=jax.ShapeDtypeStruct((_ROWS, _D), jnp.float32),
        in_specs=[
            pl.BlockSpec(memory_space=pl.ANY),
            pl.BlockSpec(memory_space=pltpu.SMEM),
            pl.BlockSpec(memory_space=pl.ANY),
        ],
        out_specs=pl.BlockSpec(memory_space=pl.ANY),
        scratch_shapes=[
            pltpu.VMEM((_NBUF, _CH, _D), jnp.float32),
            pltpu.VMEM((3, _D), jnp.float32),
            pltpu.SemaphoreType.DMA((_NBUF,)),
            pltpu.SemaphoreType.DMA((_NBUF,)),
            pltpu.SemaphoreType.DMA,
        ],
        compiler_params=pltpu.CompilerParams(
            vmem_limit_bytes=134217728,
        ),
    )(y, c, w)
